# Initial kernel scaffold; baseline (speedup 1.0000x reference)
import jax
import jax.numpy as jnp
from jax.experimental import pallas as pl


def kernel(x, edge_index, weight, bias):
    raise NotImplementedError("write your pallas kernel here")


# speedup vs baseline: 20.0078x; 20.0078x over previous
import jax
import jax.numpy as jnp
from jax import lax
from jax.experimental import pallas as pl
from jax.experimental.pallas import tpu as pltpu
from jax.experimental.pallas import tpu_sc as plsc

_N = 10000
_D = 128
_NC = 2
_NS = 16
_NW = _NC * _NS
_CH = 128
_NCH = 79
_EPT = _NCH * _CH
_EPAD = _NW * _EPT
_NHR = 80
_NH = _NHR * _D
_DUMMY = _N
_NACC = 10112
_ZPT = _NACC // _NS
_WPT = 624
_BN = 2048

_mesh = plsc.VectorSubcoreMesh(
    core_axis_name="c", subcore_axis_name="s", num_cores=_NC, num_subcores=_NS
)
_sc_params = pltpu.CompilerParams(needs_layout_passes=False)


def _s1_body(row_hbm, col_hbm, hists_hbm, ceff_hbm, row_l, col_l, ceff_l, hist_l):
    wid = lax.axis_index("s") * _NC + lax.axis_index("c")
    pltpu.sync_copy(row_hbm.at[wid], row_l)
    pltpu.sync_copy(col_hbm.at[wid], col_l)

    def zero(j, carry):
        for k in range(_D // 16):
            hist_l[j, pl.ds(k * 16, 16)] = jnp.zeros((16,), jnp.float32)
        return carry

    lax.fori_loop(0, _NHR, zero, 0)

    def step(j, carry):
        for k in range(_D // 16):
            r = row_l[j, pl.ds(k * 16, 16)]
            c = col_l[j, pl.ds(k * 16, 16)]
            m = r != c
            ew = jnp.where(m, jnp.float32(1.0), jnp.float32(0.0))
            plsc.addupdate_scatter(
                hist_l, [jnp.right_shift(c, 7), jnp.bitwise_and(c, 127)], ew
            )
            ceff_l[j, pl.ds(k * 16, 16)] = jnp.where(m, c, jnp.int32(_DUMMY))
        return carry

    lax.fori_loop(0, _NCH, step, 0)
    pltpu.sync_copy(hist_l, hists_hbm.at[wid])
    pltpu.sync_copy(ceff_l, ceff_hbm.at[wid])


_s1 = pl.kernel(
    _s1_body,
    out_type=(
        jax.ShapeDtypeStruct((_NW, _NHR, _D), jnp.float32),
        jax.ShapeDtypeStruct((_NW, _NCH, _CH), jnp.int32),
    ),
    mesh=_mesh,
    compiler_params=_sc_params,
    scratch_types=[
        pltpu.VMEM((_NCH, _CH), jnp.int32),
        pltpu.VMEM((_NCH, _CH), jnp.int32),
        pltpu.VMEM((_NCH, _CH), jnp.int32),
        pltpu.VMEM((_NHR, _D), jnp.float32),
    ],
)


def _s2_body(row_hbm, ceff_hbm, y_hbm, accs_hbm, row_l, ceff_l, gbuf, sem, acc):
    cid = lax.axis_index("c")
    sid = lax.axis_index("s")
    wid = sid * _NC + cid
    pltpu.sync_copy(row_hbm.at[wid], row_l)
    pltpu.sync_copy(ceff_hbm.at[wid], ceff_l)

    def zg(j, carry):
        for k in range(_D // 16):
            gbuf[j, pl.ds(k * 16, 16)] = jnp.zeros((16,), jnp.float32)
        return carry

    lax.fori_loop(0, _CH, zg, 0)
    zbase = sid * _ZPT
    for b in range(4):
        pltpu.sync_copy(gbuf, acc.at[pl.ds(zbase + b * _CH, _CH)])
    pltpu.sync_copy(
        gbuf.at[pl.ds(0, _ZPT - 4 * _CH)], acc.at[pl.ds(zbase + 4 * _CH, _ZPT - 4 * _CH)]
    )
    plsc.subcore_barrier()

    def chunk(i, carry):
        pltpu.async_copy(y_hbm.at[row_l.at[i]], gbuf, sem).wait()
        pltpu.sync_copy(gbuf, acc.at[ceff_l.at[i]], add=True)
        return carry

    lax.fori_loop(0, _NCH, chunk, 0)
    plsc.subcore_barrier()
    wbase = sid * _WPT
    pltpu.sync_copy(acc.at[pl.ds(wbase, _WPT)], accs_hbm.at[cid, pl.ds(wbase, _WPT)])

    @pl.when(sid == _NS - 1)
    def _tail():
        pltpu.sync_copy(
            acc.at[pl.ds(_NS * _WPT, _N - _NS * _WPT)],
            accs_hbm.at[cid, pl.ds(_NS * _WPT, _N - _NS * _WPT)],
        )


_s2 = pl.kernel(
    _s2_body,
    out_type=jax.ShapeDtypeStruct((_NC, _N, _D), jnp.float32),
    mesh=_mesh,
    compiler_params=_sc_params,
    scratch_types=[
        pltpu.VMEM((_NCH, _CH), jnp.int32),
        pltpu.VMEM((_NCH, _CH), jnp.int32),
        pltpu.VMEM((_CH, _D), jnp.float32),
        pltpu.SemaphoreType.DMA,
        pltpu.VMEM_SHARED((_NACC, _D), jnp.float32),
    ],
)


def _tca_body(hists_ref, x_ref, w_ref, y_ref):
    deg = jnp.sum(hists_ref[...], axis=0) + 1.0
    dis = lax.rsqrt(deg)
    xw = jnp.dot(x_ref[...], w_ref[...], preferred_element_type=jnp.float32)
    y_ref[...] = dis[:, None] * xw


def _tcc_body(acc_ref, y_ref, hists_ref, b_ref, o_ref):
    deg = jnp.sum(hists_ref[...], axis=0) + 1.0
    dis = lax.rsqrt(deg)
    s = acc_ref[0] + acc_ref[1] + y_ref[...]
    o_ref[...] = dis[:, None] * s + b_ref[...]


_GRID = (_NH // _BN,)

_tca = pl.pallas_call(
    _tca_body,
    grid=_GRID,
    in_specs=[
        pl.BlockSpec((_NW, _BN), lambda g: (0, g)),
        pl.BlockSpec((_BN, _D), lambda g: (g, 0)),
        pl.BlockSpec((_D, _D), lambda g: (0, 0)),
    ],
    out_specs=pl.BlockSpec((_BN, _D), lambda g: (g, 0)),
    out_shape=jax.ShapeDtypeStruct((_N, _D), jnp.float32),
)

_tcc = pl.pallas_call(
    _tcc_body,
    grid=_GRID,
    in_specs=[
        pl.BlockSpec((_NC, _BN, _D), lambda g: (0, g, 0)),
        pl.BlockSpec((_BN, _D), lambda g: (g, 0)),
        pl.BlockSpec((_NW, _BN), lambda g: (0, g)),
        pl.BlockSpec((1, _D), lambda g: (0, 0)),
    ],
    out_specs=pl.BlockSpec((_BN, _D), lambda g: (g, 0)),
    out_shape=jax.ShapeDtypeStruct((_N, _D), jnp.float32),
)


def kernel(x, edge_index, weight, bias):
    row = edge_index[0].astype(jnp.int32)
    col = edge_index[1].astype(jnp.int32)
    zpad = jnp.zeros((_EPAD - row.shape[0],), jnp.int32)
    row3 = jnp.concatenate([row, zpad]).reshape(_NW, _NCH, _CH)
    col3 = jnp.concatenate([col, zpad]).reshape(_NW, _NCH, _CH)
    hists, ceff = _s1(row3, col3)
    hists2d = hists.reshape(_NW, _NH)
    y = _tca(hists2d, x, weight)
    accs = _s2(row3, ceff, y)
    return _tcc(accs, y, hists2d, bias.reshape(1, _D))


# speedup vs baseline: 22.7952x; 1.1393x over previous
import jax
import jax.numpy as jnp
from jax import lax
from jax.experimental import pallas as pl
from jax.experimental.pallas import tpu as pltpu
from jax.experimental.pallas import tpu_sc as plsc

_N = 10000
_D = 128
_NC = 2
_NS = 16
_NW = _NC * _NS
_CH = 128
_NCH = 80
_NBUF = 4
_EPT = _NCH * _CH
_EPAD = _NW * _EPT
_NHR = 80
_NH = _NHR * _D
_DUMMY = _N
_NACC = 10112
_ZPT = _NACC // _NS
_WPT = 624
_BN = 2048

_mesh = plsc.VectorSubcoreMesh(
    core_axis_name="c", subcore_axis_name="s", num_cores=_NC, num_subcores=_NS
)
_sc_params = pltpu.CompilerParams(needs_layout_passes=False)
_sc_params_lin = pltpu.CompilerParams(
    needs_layout_passes=False, use_tc_tiling_on_sc=False
)


def _s1_body(row_hbm, col_hbm, hists_hbm, ceff_hbm, row_l, col_l, ceff_l, hist_l):
    wid = lax.axis_index("s") * _NC + lax.axis_index("c")
    pltpu.sync_copy(row_hbm.at[wid], row_l)
    pltpu.sync_copy(col_hbm.at[wid], col_l)

    def zero(j, carry):
        for k in range(_D // 16):
            hist_l[j, pl.ds(k * 16, 16)] = jnp.zeros((16,), jnp.float32)
        return carry

    lax.fori_loop(0, _NHR, zero, 0)

    def step(j, carry):
        for k in range(_D // 16):
            r = row_l[j, pl.ds(k * 16, 16)]
            c = col_l[j, pl.ds(k * 16, 16)]
            m = r != c
            ew = jnp.where(m, jnp.float32(1.0), jnp.float32(0.0))
            plsc.addupdate_scatter(
                hist_l, [jnp.right_shift(c, 7), jnp.bitwise_and(c, 127)], ew
            )
            ceff_l[j, pl.ds(k * 16, 16)] = jnp.where(m, c, jnp.int32(_DUMMY))
        return carry

    lax.fori_loop(0, _NCH, step, 0)
    pltpu.sync_copy(hist_l, hists_hbm.at[wid])
    pltpu.sync_copy(ceff_l, ceff_hbm.at[wid])


_s1 = pl.kernel(
    _s1_body,
    out_type=(
        jax.ShapeDtypeStruct((_NW, _NHR, _D), jnp.float32),
        jax.ShapeDtypeStruct((_NW, _NCH, _CH), jnp.int32),
    ),
    mesh=_mesh,
    compiler_params=_sc_params,
    scratch_types=[
        pltpu.VMEM((_NCH, _CH), jnp.int32),
        pltpu.VMEM((_NCH, _CH), jnp.int32),
        pltpu.VMEM((_NCH, _CH), jnp.int32),
        pltpu.VMEM((_NHR, _D), jnp.float32),
    ],
)


def _s2_body(
    row_hbm, ceff_hbm, y_hbm, accs_hbm, row_l, ceff_l,
    g0, g1, g2, g3, gs0, gs1, gs2, gs3, ss0, ss1, ss2, ss3, acc,
):
    g = (g0, g1, g2, g3)
    gsem = (gs0, gs1, gs2, gs3)
    ssem = (ss0, ss1, ss2, ss3)
    cid = lax.axis_index("c")
    sid = lax.axis_index("s")
    wid = sid * _NC + cid
    pltpu.sync_copy(row_hbm.at[wid], row_l)
    pltpu.sync_copy(ceff_hbm.at[wid], ceff_l)

    def zg(j, carry):
        for k in range(_D // 32):
            g0[j, pl.ds(k * 32, 32)] = jnp.zeros((32,), jnp.bfloat16)
        return carry

    lax.fori_loop(0, _CH, zg, 0)
    zbase = sid * _WPT
    for b in range(4):
        pltpu.sync_copy(g0, acc.at[pl.ds(zbase + b * _CH, _CH)])
    pltpu.sync_copy(
        g0.at[pl.ds(0, _WPT - 4 * _CH)], acc.at[pl.ds(zbase + 4 * _CH, _WPT - 4 * _CH)]
    )

    @pl.when(sid == _NS - 1)
    def _ztail():
        pltpu.sync_copy(g0, acc.at[pl.ds(_NS * _WPT, _NACC - _NS * _WPT)])

    plsc.subcore_barrier()

    for b in range(_NBUF):
        pltpu.async_copy(y_hbm.at[row_l.at[b]], g[b], gsem[b])

    def group(iog, carry):
        io = iog * _NBUF
        for b in range(_NBUF):
            i = io + b
            pltpu.make_async_copy(y_hbm.at[row_l.at[i]], g[b], gsem[b]).wait()
            pltpu.async_copy(g[b], acc.at[ceff_l.at[i]], ssem[b], add=True)
        for b in range(_NBUF):
            i = io + b
            pltpu.make_async_copy(g[b], acc.at[ceff_l.at[i]], ssem[b]).wait()
            pltpu.async_copy(y_hbm.at[row_l.at[i + _NBUF]], g[b], gsem[b])
        return carry

    lax.fori_loop(0, _NCH // _NBUF - 1, group, 0)
    for b in range(_NBUF):
        i = _NCH - _NBUF + b
        pltpu.make_async_copy(y_hbm.at[row_l.at[i]], g[b], gsem[b]).wait()
        pltpu.sync_copy(g[b], acc.at[ceff_l.at[i]], add=True)
    plsc.subcore_barrier()
    wbase = sid * _WPT
    pltpu.sync_copy(acc.at[pl.ds(wbase, _WPT)], accs_hbm.at[cid, pl.ds(wbase, _WPT)])

    @pl.when(sid == _NS - 1)
    def _tail():
        pltpu.sync_copy(
            acc.at[pl.ds(_NS * _WPT, _N - _NS * _WPT)],
            accs_hbm.at[cid, pl.ds(_NS * _WPT, _N - _NS * _WPT)],
        )


_s2 = pl.kernel(
    _s2_body,
    out_type=jax.ShapeDtypeStruct((_NC, _N, _D), jnp.bfloat16),
    mesh=_mesh,
    compiler_params=_sc_params_lin,
    scratch_types=[
        pltpu.VMEM((_NCH, _CH), jnp.int32),
        pltpu.VMEM((_NCH, _CH), jnp.int32),
    ]
    + [pltpu.VMEM((_CH, _D), jnp.bfloat16)] * _NBUF
    + [pltpu.SemaphoreType.DMA] * (2 * _NBUF)
    + [pltpu.VMEM_SHARED((_NACC, _D), jnp.bfloat16)],
)


def _tca_body(hists_ref, x_ref, w_ref, y_ref, ybf_ref):
    deg = jnp.sum(hists_ref[...], axis=0) + 1.0
    dis = lax.rsqrt(deg)
    xw = jnp.dot(x_ref[...], w_ref[...], preferred_element_type=jnp.float32)
    y = dis[:, None] * xw
    y_ref[...] = y
    ybf_ref[...] = y.astype(jnp.bfloat16)


def _tcc_body(acc_ref, y_ref, hists_ref, b_ref, o_ref):
    deg = jnp.sum(hists_ref[...], axis=0) + 1.0
    dis = lax.rsqrt(deg)
    s = (acc_ref[0] + acc_ref[1]).astype(jnp.float32) + y_ref[...]
    o_ref[...] = dis[:, None] * s + b_ref[...]


_GRID = (_NH // _BN,)

_tca = pl.pallas_call(
    _tca_body,
    grid=_GRID,
    in_specs=[
        pl.BlockSpec((_NW, _BN), lambda g: (0, g)),
        pl.BlockSpec((_BN, _D), lambda g: (g, 0)),
        pl.BlockSpec((_D, _D), lambda g: (0, 0)),
    ],
    out_specs=(
        pl.BlockSpec((_BN, _D), lambda g: (g, 0)),
        pl.BlockSpec((_BN, _D), lambda g: (g, 0)),
    ),
    out_shape=(
        jax.ShapeDtypeStruct((_N, _D), jnp.float32),
        jax.ShapeDtypeStruct((_N, _D), jnp.bfloat16),
    ),
)

_tcc = pl.pallas_call(
    _tcc_body,
    grid=_GRID,
    in_specs=[
        pl.BlockSpec((_NC, _BN, _D), lambda g: (0, g, 0)),
        pl.BlockSpec((_BN, _D), lambda g: (g, 0)),
        pl.BlockSpec((_NW, _BN), lambda g: (0, g)),
        pl.BlockSpec((1, _D), lambda g: (0, 0)),
    ],
    out_specs=pl.BlockSpec((_BN, _D), lambda g: (g, 0)),
    out_shape=jax.ShapeDtypeStruct((_N, _D), jnp.float32),
)


def kernel(x, edge_index, weight, bias):
    row = edge_index[0].astype(jnp.int32)
    col = edge_index[1].astype(jnp.int32)
    zpad = jnp.zeros((_EPAD - row.shape[0],), jnp.int32)
    row3 = jnp.concatenate([row, zpad]).reshape(_NW, _NCH, _CH)
    col3 = jnp.concatenate([col, zpad]).reshape(_NW, _NCH, _CH)
    hists, ceff = _s1(row3, col3)
    hists2d = hists.reshape(_NW, _NH)
    y, ybf = _tca(hists2d, x, weight)
    accs = _s2(row3, ceff, ybf)
    return _tcc(accs, y, hists2d, bias.reshape(1, _D))


# speedup vs baseline: 23.1669x; 1.0163x over previous
import jax
import jax.numpy as jnp
from jax import lax
from jax.experimental import pallas as pl
from jax.experimental.pallas import tpu as pltpu
from jax.experimental.pallas import tpu_sc as plsc

_N = 10000
_D = 128
_NC = 2
_NS = 16
_NW = _NC * _NS
_CH = 128
_NCH = 80
_NBUF = 4
_EPT = _NCH * _CH
_EPAD = _NW * _EPT
_NHR = 80
_NH = _NHR * _D
_DUMMY = _N
_NACC = 10112
_ZPT = _NACC // _NS
_WPT = 624
_BN = 2048

_mesh = plsc.VectorSubcoreMesh(
    core_axis_name="c", subcore_axis_name="s", num_cores=_NC, num_subcores=_NS
)
_sc_params = pltpu.CompilerParams(needs_layout_passes=False)
_sc_params_lin = pltpu.CompilerParams(
    needs_layout_passes=False, use_tc_tiling_on_sc=False
)


def _s1_body(row_hbm, col_hbm, hists_hbm, ceff_hbm, row_l, col_l, ceff_l, hist_l):
    wid = lax.axis_index("s") * _NC + lax.axis_index("c")
    pltpu.sync_copy(row_hbm.at[wid], row_l)
    pltpu.sync_copy(col_hbm.at[wid], col_l)

    def zero(j, carry):
        for k in range(_D // 16):
            hist_l[j, pl.ds(k * 16, 16)] = jnp.zeros((16,), jnp.float32)
        return carry

    lax.fori_loop(0, _NHR, zero, 0)

    lane = jax.lax.iota(jnp.int32, 16)

    def step(j, carry):
        for k in range(_CH // 16):
            r = row_l[j, pl.ds(k * 16, 16)]
            c = col_l[j, pl.ds(k * 16, 16)]
            m = r != c
            ew = jnp.where(m, jnp.float32(1.0), jnp.float32(0.0))
            dummy = _DUMMY + jnp.bitwise_and(lane + (j * (_CH // 16) + k) * 16, 63)
            ce = jnp.where(m, c, dummy)
            plsc.addupdate_scatter(
                hist_l, [jnp.right_shift(ce, 7), jnp.bitwise_and(ce, 127)], ew
            )
            ceff_l[j, pl.ds(k * 16, 16)] = ce
        return carry

    lax.fori_loop(0, _NCH, step, 0)
    pltpu.sync_copy(hist_l, hists_hbm.at[wid])
    pltpu.sync_copy(ceff_l, ceff_hbm.at[wid])


_s1 = pl.kernel(
    _s1_body,
    out_type=(
        jax.ShapeDtypeStruct((_NW, _NHR, _D), jnp.float32),
        jax.ShapeDtypeStruct((_NW, _NCH, _CH), jnp.int32),
    ),
    mesh=_mesh,
    compiler_params=_sc_params,
    scratch_types=[
        pltpu.VMEM((_NCH, _CH), jnp.int32),
        pltpu.VMEM((_NCH, _CH), jnp.int32),
        pltpu.VMEM((_NCH, _CH), jnp.int32),
        pltpu.VMEM((_NHR, _D), jnp.float32),
    ],
)


def _s2_body(
    row_hbm, ceff_hbm, y_hbm, accs_hbm, row_l, ceff_l,
    g0, g1, g2, g3, gs0, gs1, gs2, gs3, ss0, ss1, ss2, ss3, acc,
):
    g = (g0, g1, g2, g3)
    gsem = (gs0, gs1, gs2, gs3)
    ssem = (ss0, ss1, ss2, ss3)
    cid = lax.axis_index("c")
    sid = lax.axis_index("s")
    wid = sid * _NC + cid
    pltpu.sync_copy(row_hbm.at[wid], row_l)
    pltpu.sync_copy(ceff_hbm.at[wid], ceff_l)

    def zg(j, carry):
        for k in range(_D // 32):
            g0[j, pl.ds(k * 32, 32)] = jnp.zeros((32,), jnp.bfloat16)
        return carry

    lax.fori_loop(0, _CH, zg, 0)
    zbase = sid * _WPT
    for b in range(4):
        pltpu.sync_copy(g0, acc.at[pl.ds(zbase + b * _CH, _CH)])
    pltpu.sync_copy(
        g0.at[pl.ds(0, _WPT - 4 * _CH)], acc.at[pl.ds(zbase + 4 * _CH, _WPT - 4 * _CH)]
    )

    @pl.when(sid == _NS - 1)
    def _ztail():
        pltpu.sync_copy(g0, acc.at[pl.ds(_NS * _WPT, _NACC - _NS * _WPT)])

    plsc.subcore_barrier()

    for b in range(_NBUF):
        pltpu.async_copy(y_hbm.at[row_l.at[b]], g[b], gsem[b])

    def group(iog, carry):
        io = iog * _NBUF
        for b in range(_NBUF):
            i = io + b
            pltpu.make_async_copy(y_hbm.at[row_l.at[i]], g[b], gsem[b]).wait()
            pltpu.async_copy(g[b], acc.at[ceff_l.at[i]], ssem[b], add=True)
        for b in range(_NBUF):
            i = io + b
            pltpu.make_async_copy(g[b], acc.at[ceff_l.at[i]], ssem[b]).wait()
            pltpu.async_copy(y_hbm.at[row_l.at[i + _NBUF]], g[b], gsem[b])
        return carry

    lax.fori_loop(0, _NCH // _NBUF - 1, group, 0)
    for b in range(_NBUF):
        i = _NCH - _NBUF + b
        pltpu.make_async_copy(y_hbm.at[row_l.at[i]], g[b], gsem[b]).wait()
        pltpu.sync_copy(g[b], acc.at[ceff_l.at[i]], add=True)
    plsc.subcore_barrier()
    wbase = sid * _WPT
    pltpu.sync_copy(acc.at[pl.ds(wbase, _WPT)], accs_hbm.at[cid, pl.ds(wbase, _WPT)])

    @pl.when(sid == _NS - 1)
    def _tail():
        pltpu.sync_copy(
            acc.at[pl.ds(_NS * _WPT, _N - _NS * _WPT)],
            accs_hbm.at[cid, pl.ds(_NS * _WPT, _N - _NS * _WPT)],
        )


_s2 = pl.kernel(
    _s2_body,
    out_type=jax.ShapeDtypeStruct((_NC, _N, _D), jnp.bfloat16),
    mesh=_mesh,
    compiler_params=_sc_params_lin,
    scratch_types=[
        pltpu.VMEM((_NCH, _CH), jnp.int32),
        pltpu.VMEM((_NCH, _CH), jnp.int32),
    ]
    + [pltpu.VMEM((_CH, _D), jnp.bfloat16)] * _NBUF
    + [pltpu.SemaphoreType.DMA] * (2 * _NBUF)
    + [pltpu.VMEM_SHARED((_NACC, _D), jnp.bfloat16)],
)


def _tca_body(hists_ref, x_ref, w_ref, y_ref, ybf_ref):
    deg = jnp.sum(hists_ref[...], axis=0) + 1.0
    dis = lax.rsqrt(deg)
    xw = jnp.dot(x_ref[...], w_ref[...], preferred_element_type=jnp.float32)
    y = dis[:, None] * xw
    y_ref[...] = y
    ybf_ref[...] = y.astype(jnp.bfloat16)


def _tcc_body(acc_ref, y_ref, hists_ref, b_ref, o_ref):
    deg = jnp.sum(hists_ref[...], axis=0) + 1.0
    dis = lax.rsqrt(deg)
    s = (acc_ref[0] + acc_ref[1]).astype(jnp.float32) + y_ref[...]
    o_ref[...] = dis[:, None] * s + b_ref[...]


_GRID = (_NH // _BN,)

_tca = pl.pallas_call(
    _tca_body,
    grid=_GRID,
    in_specs=[
        pl.BlockSpec((_NW, _BN), lambda g: (0, g)),
        pl.BlockSpec((_BN, _D), lambda g: (g, 0)),
        pl.BlockSpec((_D, _D), lambda g: (0, 0)),
    ],
    out_specs=(
        pl.BlockSpec((_BN, _D), lambda g: (g, 0)),
        pl.BlockSpec((_BN, _D), lambda g: (g, 0)),
    ),
    out_shape=(
        jax.ShapeDtypeStruct((_N, _D), jnp.float32),
        jax.ShapeDtypeStruct((_N, _D), jnp.bfloat16),
    ),
)

_tcc = pl.pallas_call(
    _tcc_body,
    grid=_GRID,
    in_specs=[
        pl.BlockSpec((_NC, _BN, _D), lambda g: (0, g, 0)),
        pl.BlockSpec((_BN, _D), lambda g: (g, 0)),
        pl.BlockSpec((_NW, _BN), lambda g: (0, g)),
        pl.BlockSpec((1, _D), lambda g: (0, 0)),
    ],
    out_specs=pl.BlockSpec((_BN, _D), lambda g: (g, 0)),
    out_shape=jax.ShapeDtypeStruct((_N, _D), jnp.float32),
)


def kernel(x, edge_index, weight, bias):
    row = edge_index[0].astype(jnp.int32)
    col = edge_index[1].astype(jnp.int32)
    zpad = jnp.zeros((_EPAD - row.shape[0],), jnp.int32)
    row3 = jnp.concatenate([row, zpad]).reshape(_NW, _NCH, _CH)
    col3 = jnp.concatenate([col, zpad]).reshape(_NW, _NCH, _CH)
    hists, ceff = _s1(row3, col3)
    hists2d = hists.reshape(_NW, _NH)
    y, ybf = _tca(hists2d, x, weight)
    accs = _s2(row3, ceff, ybf)
    return _tcc(accs, y, hists2d, bias.reshape(1, _D))


# speedup vs baseline: 39.7733x; 1.7168x over previous
import jax
import jax.numpy as jnp
from jax import lax
from jax.experimental import pallas as pl
from jax.experimental.pallas import tpu as pltpu
from jax.experimental.pallas import tpu_sc as plsc

_N = 10000
_D = 128
_NC = 2
_NS = 16
_NW = _NC * _NS
_CH = 128
_NCH = 81
_NBUF = 3
_EPT = _NCH * _CH
_EPAD = _NW * _EPT
_NHR = 80
_NH = _NHR * _D
_DUMMY = _N
_NACC = 10112
_ZPT = _NACC // _NS
_WPT = 624
_BN = 2048

_mesh = plsc.VectorSubcoreMesh(
    core_axis_name="c", subcore_axis_name="s", num_cores=_NC, num_subcores=_NS
)
_sc_params = pltpu.CompilerParams(needs_layout_passes=False)
_sc_params_lin = pltpu.CompilerParams(
    needs_layout_passes=False, use_tc_tiling_on_sc=False
)


def _s1_body(row_hbm, col_hbm, hists_hbm, ceff_hbm, row_l, col_l, ceff_l, hist_l):
    wid = lax.axis_index("s") * _NC + lax.axis_index("c")
    pltpu.sync_copy(row_hbm.at[wid], row_l)
    pltpu.sync_copy(col_hbm.at[wid], col_l)

    def zero(j, carry):
        for k in range(_D // 16):
            hist_l[j, pl.ds(k * 16, 16)] = jnp.zeros((16,), jnp.float32)
        return carry

    lax.fori_loop(0, _NHR, zero, 0)

    lane = jax.lax.iota(jnp.int32, 16)

    def step(j, carry):
        for k in range(_CH // 16):
            r = row_l[j, pl.ds(k * 16, 16)]
            c = col_l[j, pl.ds(k * 16, 16)]
            m = r != c
            ew = jnp.where(m, jnp.float32(1.0), jnp.float32(0.0))
            dummy = _DUMMY + jnp.bitwise_and(lane + (j * (_CH // 16) + k) * 16, 63)
            ce = jnp.where(m, c, dummy)
            plsc.addupdate_scatter(
                hist_l, [jnp.right_shift(ce, 7), jnp.bitwise_and(ce, 127)], ew
            )
            ceff_l[j, pl.ds(k * 16, 16)] = ce
        return carry

    lax.fori_loop(0, _NCH, step, 0)
    pltpu.sync_copy(hist_l, hists_hbm.at[wid])
    pltpu.sync_copy(ceff_l, ceff_hbm.at[wid])


_s1 = pl.kernel(
    _s1_body,
    out_type=(
        jax.ShapeDtypeStruct((_NW, _NHR, _D), jnp.float32),
        jax.ShapeDtypeStruct((_NW, _NCH, _CH), jnp.int32),
    ),
    mesh=_mesh,
    compiler_params=_sc_params,
    scratch_types=[
        pltpu.VMEM((_NCH, _CH), jnp.int32),
        pltpu.VMEM((_NCH, _CH), jnp.int32),
        pltpu.VMEM((_NCH, _CH), jnp.int32),
        pltpu.VMEM((_NHR, _D), jnp.float32),
    ],
)


def _s2_body(
    row_hbm, ceff_hbm, y_hbm, accs_hbm, row_l, ceff_l,
    g0, g1, g2, gs0, gs1, gs2, ss0, ss1, ss2, ysem, acc, y_sh,
):
    g = (g0, g1, g2)
    gsem = (gs0, gs1, gs2)
    ssem = (ss0, ss1, ss2)
    cid = lax.axis_index("c")
    sid = lax.axis_index("s")
    wid = sid * _NC + cid
    ycopy = pltpu.async_copy(
        y_hbm.at[pl.ds(sid * _WPT, _WPT)], y_sh.at[pl.ds(sid * _WPT, _WPT)], ysem
    )
    pltpu.sync_copy(row_hbm.at[wid], row_l)
    pltpu.sync_copy(ceff_hbm.at[wid], ceff_l)

    def zg(j, carry):
        for k in range(_D // 32):
            g0[j, pl.ds(k * 32, 32)] = jnp.zeros((32,), jnp.bfloat16)
        return carry

    lax.fori_loop(0, _CH, zg, 0)
    zbase = sid * _WPT
    for b in range(4):
        pltpu.sync_copy(g0, acc.at[pl.ds(zbase + b * _CH, _CH)])
    pltpu.sync_copy(
        g0.at[pl.ds(0, _WPT - 4 * _CH)], acc.at[pl.ds(zbase + 4 * _CH, _WPT - 4 * _CH)]
    )

    @pl.when(sid == _NS - 1)
    def _ztail():
        pltpu.sync_copy(g0, acc.at[pl.ds(_NS * _WPT, _NACC - _NS * _WPT)])
        pltpu.sync_copy(
            y_hbm.at[pl.ds(_NS * _WPT, _N - _NS * _WPT)],
            y_sh.at[pl.ds(_NS * _WPT, _N - _NS * _WPT)],
        )

    ycopy.wait()
    plsc.subcore_barrier()

    for b in range(_NBUF):
        pltpu.async_copy(y_sh.at[row_l.at[b]], g[b], gsem[b])

    def group(iog, carry):
        io = iog * _NBUF
        for b in range(_NBUF):
            i = io + b
            pltpu.make_async_copy(y_sh.at[row_l.at[i]], g[b], gsem[b]).wait()
            pltpu.async_copy(g[b], acc.at[ceff_l.at[i]], ssem[b], add=True)
        for b in range(_NBUF):
            i = io + b
            pltpu.make_async_copy(g[b], acc.at[ceff_l.at[i]], ssem[b]).wait()
            pltpu.async_copy(y_sh.at[row_l.at[i + _NBUF]], g[b], gsem[b])
        return carry

    lax.fori_loop(0, _NCH // _NBUF - 1, group, 0)
    for b in range(_NBUF):
        i = _NCH - _NBUF + b
        pltpu.make_async_copy(y_sh.at[row_l.at[i]], g[b], gsem[b]).wait()
        pltpu.sync_copy(g[b], acc.at[ceff_l.at[i]], add=True)
    plsc.subcore_barrier()
    wbase = sid * _WPT
    pltpu.sync_copy(acc.at[pl.ds(wbase, _WPT)], accs_hbm.at[cid, pl.ds(wbase, _WPT)])

    @pl.when(sid == _NS - 1)
    def _tail():
        pltpu.sync_copy(
            acc.at[pl.ds(_NS * _WPT, _N - _NS * _WPT)],
            accs_hbm.at[cid, pl.ds(_NS * _WPT, _N - _NS * _WPT)],
        )


_s2 = pl.kernel(
    _s2_body,
    out_type=jax.ShapeDtypeStruct((_NC, _N, _D), jnp.bfloat16),
    mesh=_mesh,
    compiler_params=_sc_params_lin,
    scratch_types=[
        pltpu.VMEM((_NCH, _CH), jnp.int32),
        pltpu.VMEM((_NCH, _CH), jnp.int32),
    ]
    + [pltpu.VMEM((_CH, _D), jnp.bfloat16)] * _NBUF
    + [pltpu.SemaphoreType.DMA] * (2 * _NBUF + 1)
    + [
        pltpu.VMEM_SHARED((_NACC, _D), jnp.bfloat16),
        pltpu.VMEM_SHARED((_N, _D), jnp.bfloat16),
    ],
)


def _tca_body(hists_ref, x_ref, w_ref, y_ref, ybf_ref):
    deg = jnp.sum(hists_ref[...], axis=0) + 1.0
    dis = lax.rsqrt(deg)
    xw = jnp.dot(x_ref[...], w_ref[...], preferred_element_type=jnp.float32)
    y = dis[:, None] * xw
    y_ref[...] = y
    ybf_ref[...] = y.astype(jnp.bfloat16)


def _tcc_body(acc_ref, y_ref, hists_ref, b_ref, o_ref):
    deg = jnp.sum(hists_ref[...], axis=0) + 1.0
    dis = lax.rsqrt(deg)
    s = (acc_ref[0] + acc_ref[1]).astype(jnp.float32) + y_ref[...]
    o_ref[...] = dis[:, None] * s + b_ref[...]


_GRID = (_NH // _BN,)

_tca = pl.pallas_call(
    _tca_body,
    grid=_GRID,
    in_specs=[
        pl.BlockSpec((_NW, _BN), lambda g: (0, g)),
        pl.BlockSpec((_BN, _D), lambda g: (g, 0)),
        pl.BlockSpec((_D, _D), lambda g: (0, 0)),
    ],
    out_specs=(
        pl.BlockSpec((_BN, _D), lambda g: (g, 0)),
        pl.BlockSpec((_BN, _D), lambda g: (g, 0)),
    ),
    out_shape=(
        jax.ShapeDtypeStruct((_N, _D), jnp.float32),
        jax.ShapeDtypeStruct((_N, _D), jnp.bfloat16),
    ),
)

_tcc = pl.pallas_call(
    _tcc_body,
    grid=_GRID,
    in_specs=[
        pl.BlockSpec((_NC, _BN, _D), lambda g: (0, g, 0)),
        pl.BlockSpec((_BN, _D), lambda g: (g, 0)),
        pl.BlockSpec((_NW, _BN), lambda g: (0, g)),
        pl.BlockSpec((1, _D), lambda g: (0, 0)),
    ],
    out_specs=pl.BlockSpec((_BN, _D), lambda g: (g, 0)),
    out_shape=jax.ShapeDtypeStruct((_N, _D), jnp.float32),
)


def kernel(x, edge_index, weight, bias):
    row = edge_index[0].astype(jnp.int32)
    col = edge_index[1].astype(jnp.int32)
    zpad = jnp.zeros((_EPAD - row.shape[0],), jnp.int32)
    row3 = jnp.concatenate([row, zpad]).reshape(_NW, _NCH, _CH)
    col3 = jnp.concatenate([col, zpad]).reshape(_NW, _NCH, _CH)
    hists, ceff = _s1(row3, col3)
    hists2d = hists.reshape(_NW, _NH)
    y, ybf = _tca(hists2d, x, weight)
    accs = _s2(row3, ceff, ybf)
    return _tcc(accs, y, hists2d, bias.reshape(1, _D))


# speedup vs baseline: 40.1184x; 1.0087x over previous
import jax
import jax.numpy as jnp
from jax import lax
from jax.experimental import pallas as pl
from jax.experimental.pallas import tpu as pltpu
from jax.experimental.pallas import tpu_sc as plsc

_N = 10000
_D = 128
_NC = 2
_NS = 16
_NW = _NC * _NS
_CH = 128
_NCH = 81
_NBUF = 3
_EPT = _NCH * _CH
_EPAD = _NW * _EPT
_NHR = 80
_NH = _NHR * _D
_DUMMY = _N
_NACC = 10112
_ZPT = _NACC // _NS
_WPT = 624
_BN = 2048

_mesh = plsc.VectorSubcoreMesh(
    core_axis_name="c", subcore_axis_name="s", num_cores=_NC, num_subcores=_NS
)
_sc_params = pltpu.CompilerParams(needs_layout_passes=False)
_sc_params_lin = pltpu.CompilerParams(
    needs_layout_passes=False, use_tc_tiling_on_sc=False
)


def _s1_body(row_hbm, col_hbm, hists_hbm, ceff_hbm, row_l, col_l, ceff_l, hist_l):
    wid = lax.axis_index("s") * _NC + lax.axis_index("c")
    pltpu.sync_copy(row_hbm.at[wid], row_l)
    pltpu.sync_copy(col_hbm.at[wid], col_l)

    def zero(j, carry):
        for k in range(_D // 16):
            hist_l[j, pl.ds(k * 16, 16)] = jnp.zeros((16,), jnp.float32)
        return carry

    lax.fori_loop(0, _NHR, zero, 0)

    lane = jax.lax.iota(jnp.int32, 16)

    def step(j, carry):
        for k in range(_CH // 16):
            r = row_l[j, pl.ds(k * 16, 16)]
            c = col_l[j, pl.ds(k * 16, 16)]
            m = r != c
            ew = jnp.where(m, jnp.float32(1.0), jnp.float32(0.0))
            dummy = _DUMMY + jnp.bitwise_and(lane + (j * (_CH // 16) + k) * 16, 63)
            ce = jnp.where(m, c, dummy)
            plsc.addupdate_scatter(
                hist_l, [jnp.right_shift(ce, 7), jnp.bitwise_and(ce, 127)], ew
            )
            ceff_l[j, pl.ds(k * 16, 16)] = ce
        return carry

    lax.fori_loop(0, _NCH, step, 0)
    pltpu.sync_copy(hist_l, hists_hbm.at[wid])
    pltpu.sync_copy(ceff_l, ceff_hbm.at[wid])


_s1 = pl.kernel(
    _s1_body,
    out_type=(
        jax.ShapeDtypeStruct((_NW, _NHR, _D), jnp.float32),
        jax.ShapeDtypeStruct((_NW, _NCH, _CH), jnp.int32),
    ),
    mesh=_mesh,
    compiler_params=_sc_params_lin,
    scratch_types=[
        pltpu.VMEM((_NCH, _CH), jnp.int32),
        pltpu.VMEM((_NCH, _CH), jnp.int32),
        pltpu.VMEM((_NCH, _CH), jnp.int32),
        pltpu.VMEM((_NHR, _D), jnp.float32),
    ],
)


def _s2_body(
    row_hbm, ceff_hbm, y_hbm, accs_hbm, row_l, ceff_l,
    g0, g1, g2, gs0, gs1, gs2, ss0, ss1, ss2, ysem, acc, y_sh,
):
    g = (g0, g1, g2)
    gsem = (gs0, gs1, gs2)
    ssem = (ss0, ss1, ss2)
    cid = lax.axis_index("c")
    sid = lax.axis_index("s")
    wid = sid * _NC + cid
    ycopy = pltpu.async_copy(
        y_hbm.at[pl.ds(sid * _WPT, _WPT)], y_sh.at[pl.ds(sid * _WPT, _WPT)], ysem
    )
    pltpu.sync_copy(row_hbm.at[wid], row_l)
    pltpu.sync_copy(ceff_hbm.at[wid], ceff_l)

    def zg(j, carry):
        for k in range(_D // 32):
            g0[j, pl.ds(k * 32, 32)] = jnp.zeros((32,), jnp.bfloat16)
        return carry

    lax.fori_loop(0, _CH, zg, 0)
    zbase = sid * _WPT
    for b in range(4):
        pltpu.sync_copy(g0, acc.at[pl.ds(zbase + b * _CH, _CH)])
    pltpu.sync_copy(
        g0.at[pl.ds(0, _WPT - 4 * _CH)], acc.at[pl.ds(zbase + 4 * _CH, _WPT - 4 * _CH)]
    )

    @pl.when(sid == _NS - 1)
    def _ztail():
        pltpu.sync_copy(g0, acc.at[pl.ds(_NS * _WPT, _NACC - _NS * _WPT)])
        pltpu.sync_copy(
            y_hbm.at[pl.ds(_NS * _WPT, _N - _NS * _WPT)],
            y_sh.at[pl.ds(_NS * _WPT, _N - _NS * _WPT)],
        )

    ycopy.wait()
    plsc.subcore_barrier()

    for b in range(_NBUF):
        pltpu.async_copy(y_sh.at[row_l.at[b]], g[b], gsem[b])

    def group(iog, carry):
        io = iog * _NBUF
        for b in range(_NBUF):
            i = io + b
            pltpu.make_async_copy(y_sh.at[row_l.at[i]], g[b], gsem[b]).wait()
            pltpu.async_copy(g[b], acc.at[ceff_l.at[i]], ssem[b], add=True)
        for b in range(_NBUF):
            i = io + b
            pltpu.make_async_copy(g[b], acc.at[ceff_l.at[i]], ssem[b]).wait()
            pltpu.async_copy(y_sh.at[row_l.at[i + _NBUF]], g[b], gsem[b])
        return carry

    lax.fori_loop(0, _NCH // _NBUF - 1, group, 0)
    for b in range(_NBUF):
        i = _NCH - _NBUF + b
        pltpu.make_async_copy(y_sh.at[row_l.at[i]], g[b], gsem[b]).wait()
        pltpu.sync_copy(g[b], acc.at[ceff_l.at[i]], add=True)
    plsc.subcore_barrier()
    wbase = sid * _WPT
    pltpu.sync_copy(acc.at[pl.ds(wbase, _WPT)], accs_hbm.at[cid, pl.ds(wbase, _WPT)])

    @pl.when(sid == _NS - 1)
    def _tail():
        pltpu.sync_copy(
            acc.at[pl.ds(_NS * _WPT, _N - _NS * _WPT)],
            accs_hbm.at[cid, pl.ds(_NS * _WPT, _N - _NS * _WPT)],
        )


_s2 = pl.kernel(
    _s2_body,
    out_type=jax.ShapeDtypeStruct((_NC, _N, _D), jnp.bfloat16),
    mesh=_mesh,
    compiler_params=_sc_params_lin,
    scratch_types=[
        pltpu.VMEM((_NCH, _CH), jnp.int32),
        pltpu.VMEM((_NCH, _CH), jnp.int32),
    ]
    + [pltpu.VMEM((_CH, _D), jnp.bfloat16)] * _NBUF
    + [pltpu.SemaphoreType.DMA] * (2 * _NBUF + 1)
    + [
        pltpu.VMEM_SHARED((_NACC, _D), jnp.bfloat16),
        pltpu.VMEM_SHARED((_N, _D), jnp.bfloat16),
    ],
)


def _tca_body(hists_ref, x_ref, w_ref, y_ref, ybf_ref):
    deg = jnp.sum(hists_ref[...], axis=0) + 1.0
    dis = lax.rsqrt(deg)
    xw = jnp.dot(x_ref[...], w_ref[...], preferred_element_type=jnp.float32)
    y = dis[:, None] * xw
    y_ref[...] = y
    ybf_ref[...] = y.astype(jnp.bfloat16)


def _tcc_body(acc_ref, y_ref, hists_ref, b_ref, o_ref):
    deg = jnp.sum(hists_ref[...], axis=0) + 1.0
    dis = lax.rsqrt(deg)
    s = (acc_ref[0] + acc_ref[1]).astype(jnp.float32) + y_ref[...]
    o_ref[...] = dis[:, None] * s + b_ref[...]


_GRID = (_NH // _BN,)

_tca = pl.pallas_call(
    _tca_body,
    grid=_GRID,
    in_specs=[
        pl.BlockSpec((_NW, _BN), lambda g: (0, g)),
        pl.BlockSpec((_BN, _D), lambda g: (g, 0)),
        pl.BlockSpec((_D, _D), lambda g: (0, 0)),
    ],
    out_specs=(
        pl.BlockSpec((_BN, _D), lambda g: (g, 0)),
        pl.BlockSpec((_BN, _D), lambda g: (g, 0)),
    ),
    out_shape=(
        jax.ShapeDtypeStruct((_N, _D), jnp.float32),
        jax.ShapeDtypeStruct((_N, _D), jnp.bfloat16),
    ),
)

_tcc = pl.pallas_call(
    _tcc_body,
    grid=_GRID,
    in_specs=[
        pl.BlockSpec((_NC, _BN, _D), lambda g: (0, g, 0)),
        pl.BlockSpec((_BN, _D), lambda g: (g, 0)),
        pl.BlockSpec((_NW, _BN), lambda g: (0, g)),
        pl.BlockSpec((1, _D), lambda g: (0, 0)),
    ],
    out_specs=pl.BlockSpec((_BN, _D), lambda g: (g, 0)),
    out_shape=jax.ShapeDtypeStruct((_N, _D), jnp.float32),
)


def kernel(x, edge_index, weight, bias):
    row = edge_index[0].astype(jnp.int32)
    col = edge_index[1].astype(jnp.int32)
    zpad = jnp.zeros((_EPAD - row.shape[0],), jnp.int32)
    row3 = jnp.concatenate([row, zpad]).reshape(_NW, _NCH, _CH)
    col3 = jnp.concatenate([col, zpad]).reshape(_NW, _NCH, _CH)
    hists, ceff = _s1(row3, col3)
    hists2d = hists.reshape(_NW, _NH)
    y, ybf = _tca(hists2d, x, weight)
    accs = _s2(row3, ceff, ybf)
    return _tcc(accs, y, hists2d, bias.reshape(1, _D))
